# Initial kernel scaffold; baseline (speedup 1.0000x reference)
import jax
import jax.numpy as jnp
from jax.experimental import pallas as pl


def kernel(x, edge_index, W_self_0, W_neigh_0, b_0, W_self_1, W_neigh_1, b_1, W_self_2, W_neigh_2, b_2):
    raise NotImplementedError("write your pallas kernel here")


# speedup vs baseline: 6.9849x; 6.9849x over previous
import functools

import jax
import jax.numpy as jnp
from jax import lax
from jax.experimental import pallas as pl
from jax.experimental.pallas import tpu as pltpu
from jax.experimental.pallas import tpu_sc as plsc

_N = 10000
_D = 128
_E = 320000
_NC = 2
_NS = 16
_NW = _NC * _NS
_EPW = _E // _NW
_CH = 80
_NCHUNK = _EPW // _CH
_NP = 10240
_RPT = _NP // _NS
_DEGW = 16

_mesh = plsc.VectorSubcoreMesh(core_axis_name="c", subcore_axis_name="s")


@functools.partial(
    pl.kernel,
    out_type=jax.ShapeDtypeStruct((_NC, _NP, _D), jnp.float32),
    mesh=_mesh,
    scratch_types=[
        pltpu.VMEM((_NCHUNK, _CH), jnp.int32),
        pltpu.VMEM((_NCHUNK, _CH), jnp.int32),
        pltpu.VMEM((_CH, _D), jnp.float32),
        pltpu.VMEM_SHARED((_NP, _D), jnp.float32),
        pltpu.SemaphoreType.DMA,
    ],
)
def _sc_agg(h_hbm, src_hbm, dst_hbm, zero_hbm, out_hbm,
            srcv, dstv, rows, aggsh, sem):
    c = lax.axis_index("c")
    s = lax.axis_index("s")
    wid = c * _NS + s
    r0 = s * _RPT
    pltpu.sync_copy(zero_hbm.at[pl.ds(r0, _RPT)], aggsh.at[pl.ds(r0, _RPT)])
    pltpu.sync_copy(src_hbm.at[wid], srcv)
    pltpu.sync_copy(dst_hbm.at[wid], dstv)
    plsc.subcore_barrier()

    def step(j, carry):
        pltpu.async_copy(h_hbm.at[srcv.at[j]], rows, sem).wait()
        pltpu.sync_copy(rows, aggsh.at[dstv.at[j]], add=True)
        return carry

    lax.fori_loop(0, _NCHUNK, step, 0)
    plsc.subcore_barrier()
    pltpu.sync_copy(aggsh.at[pl.ds(r0, _RPT)], out_hbm.at[c].at[pl.ds(r0, _RPT)])


@functools.partial(
    pl.kernel,
    out_type=jax.ShapeDtypeStruct((_NC, _NP, _D), jnp.float32),
    mesh=_mesh,
    scratch_types=[
        pltpu.VMEM((_NCHUNK, _CH), jnp.int32),
        pltpu.VMEM((_CH, _D), jnp.float32),
        pltpu.VMEM_SHARED((_NP, _D), jnp.float32),
    ],
)
def _sc_deg(dst_hbm, ones_hbm, zerod_hbm, out_hbm, dstv, ones, degsh):
    c = lax.axis_index("c")
    s = lax.axis_index("s")
    wid = c * _NS + s
    r0 = s * _RPT
    pltpu.sync_copy(zerod_hbm.at[pl.ds(r0, _RPT)], degsh.at[pl.ds(r0, _RPT)])
    pltpu.sync_copy(dst_hbm.at[wid], dstv)
    pltpu.sync_copy(ones_hbm, ones)
    plsc.subcore_barrier()

    def step(j, carry):
        pltpu.sync_copy(ones, degsh.at[dstv.at[j]], add=True)
        return carry

    lax.fori_loop(0, _NCHUNK, step, 0)
    plsc.subcore_barrier()
    pltpu.sync_copy(degsh.at[pl.ds(r0, _RPT)], out_hbm.at[c].at[pl.ds(r0, _RPT)])


_BLK = 1000


def _tc_body(h_ref, p_ref, d_ref, ws_ref, wn_ref, b_ref, o_ref):
    deg = d_ref[0, :, 0:1] + d_ref[1, :, 0:1]
    inv = 1.0 / jnp.maximum(deg, 1.0)
    mean = (p_ref[0] + p_ref[1]) * inv
    acc = jnp.dot(h_ref[...], ws_ref[...], preferred_element_type=jnp.float32)
    acc = acc + jnp.dot(mean, wn_ref[...], preferred_element_type=jnp.float32)
    o_ref[...] = jnp.maximum(acc + b_ref[...], 0.0)


_tc_layer = pl.pallas_call(
    _tc_body,
    grid=(_N // _BLK,),
    in_specs=[
        pl.BlockSpec((_BLK, _D), lambda i: (i, 0)),
        pl.BlockSpec((_NC, _BLK, _D), lambda i: (0, i, 0)),
        pl.BlockSpec((_NC, _BLK, _D), lambda i: (0, i, 0)),
        pl.BlockSpec((_D, _D), lambda i: (0, 0)),
        pl.BlockSpec((_D, _D), lambda i: (0, 0)),
        pl.BlockSpec((1, _D), lambda i: (0, 0)),
    ],
    out_specs=pl.BlockSpec((_BLK, _D), lambda i: (i, 0)),
    out_shape=jax.ShapeDtypeStruct((_N, _D), jnp.float32),
)


def kernel(x, edge_index, W_self_0, W_neigh_0, b_0, W_self_1, W_neigh_1, b_1,
           W_self_2, W_neigh_2, b_2):
    src = edge_index[0].reshape(_NW, _NCHUNK, _CH)
    dst = edge_index[1].reshape(_NW, _NCHUNK, _CH)
    zeros = jnp.zeros((_NP, _D), jnp.float32)
    ones = jnp.ones((_CH, _D), jnp.float32)

    deg16 = _sc_deg(dst, ones, zeros)
    params = [(W_self_0, W_neigh_0, b_0), (W_self_1, W_neigh_1, b_1),
              (W_self_2, W_neigh_2, b_2)]
    h = x
    for Ws, Wn, b in params:
        parts = _sc_agg(h, src, dst, zeros)
        h = _tc_layer(h, parts, deg16, Ws, Wn, b.reshape(1, _D))
    return h.reshape(1, _N, _D)


# speedup vs baseline: 9.8410x; 1.4089x over previous
import functools

import jax
import jax.numpy as jnp
from jax import lax
from jax.experimental import pallas as pl
from jax.experimental.pallas import tpu as pltpu
from jax.experimental.pallas import tpu_sc as plsc

_N = 10000
_D = 128
_E = 320000
_NC = 2
_NS = 16
_NW = _NC * _NS
_EPW = _E // _NW
_CH = 128
_NCHUNK = 80
_PAD = _NCHUNK * _CH - _EPW
_SB = 40
_NSB = _NCHUNK // _SB
_NP = 10240
_RPT = _NP // _NS

_mesh = plsc.VectorSubcoreMesh(core_axis_name="c", subcore_axis_name="s")

_scratch = [
    pltpu.VMEM((_SB, _CH), jnp.int32),
    pltpu.VMEM((_SB, _CH), jnp.int32),
    pltpu.VMEM((_CH, _D), jnp.float32),
    pltpu.VMEM((_CH, _D), jnp.float32),
    pltpu.VMEM_SHARED((_NP, _D), jnp.float32),
    pltpu.SemaphoreType.DMA,
    pltpu.SemaphoreType.DMA,
]


def _edge_pass(h_hbm, src_hbm, dst_hbm, wid, srcb, dstb, rows, sems, aggsh):
    for sb in range(_NSB):
        pltpu.sync_copy(src_hbm.at[wid, pl.ds(sb * _SB, _SB)], srcb)
        pltpu.sync_copy(dst_hbm.at[wid, pl.ds(sb * _SB, _SB)], dstb)
        pltpu.async_copy(h_hbm.at[srcb.at[0]], rows[0], sems[0])

        def step(g, carry):
            for b in range(2):
                j = 2 * g + b
                pltpu.make_async_copy(h_hbm.at[srcb.at[0]], rows[b],
                                      sems[b]).wait()

                @pl.when(j + 1 < _SB)
                def _():
                    pltpu.async_copy(h_hbm.at[srcb.at[j + 1]], rows[1 - b],
                                     sems[1 - b])

                pltpu.sync_copy(rows[b], aggsh.at[dstb.at[j]], add=True)
            return carry

        lax.fori_loop(0, _SB // 2, step, 0)


@functools.partial(
    pl.kernel,
    out_type=jax.ShapeDtypeStruct((_NC, _NP, _D), jnp.float32),
    mesh=_mesh,
    scratch_types=_scratch,
)
def _sc_agg(h_hbm, src_hbm, dst_hbm, zero_hbm, out_hbm,
            srcb, dstb, rows0, rows1, aggsh, sem0, sem1):
    c = lax.axis_index("c")
    s = lax.axis_index("s")
    wid = c * _NS + s
    r0 = s * _RPT
    pltpu.sync_copy(zero_hbm.at[pl.ds(r0, _RPT)], aggsh.at[pl.ds(r0, _RPT)])
    plsc.subcore_barrier()
    _edge_pass(h_hbm, src_hbm, dst_hbm, wid, srcb, dstb, (rows0, rows1),
               (sem0, sem1), aggsh)
    plsc.subcore_barrier()
    pltpu.sync_copy(aggsh.at[pl.ds(r0, _RPT)], out_hbm.at[c].at[pl.ds(r0, _RPT)])


@functools.partial(
    pl.kernel,
    out_type=(jax.ShapeDtypeStruct((_NC, _NP, _D), jnp.float32),
              jax.ShapeDtypeStruct((_NC, _NP, _D), jnp.float32)),
    mesh=_mesh,
    scratch_types=_scratch,
)
def _sc_agg0(h_hbm, src_hbm, dst_hbm, zero_hbm, ones_hbm, out_hbm, deg_hbm,
             srcb, dstb, rows0, rows1, aggsh, sem0, sem1):
    c = lax.axis_index("c")
    s = lax.axis_index("s")
    wid = c * _NS + s
    r0 = s * _RPT
    pltpu.sync_copy(zero_hbm.at[pl.ds(r0, _RPT)], aggsh.at[pl.ds(r0, _RPT)])
    plsc.subcore_barrier()
    _edge_pass(h_hbm, src_hbm, dst_hbm, wid, srcb, dstb, (rows0, rows1),
               (sem0, sem1), aggsh)
    plsc.subcore_barrier()
    pltpu.sync_copy(aggsh.at[pl.ds(r0, _RPT)], out_hbm.at[c].at[pl.ds(r0, _RPT)])
    pltpu.sync_copy(zero_hbm.at[pl.ds(r0, _RPT)], aggsh.at[pl.ds(r0, _RPT)])
    pltpu.sync_copy(ones_hbm, rows0)
    plsc.subcore_barrier()
    for sb in range(_NSB):
        pltpu.sync_copy(dst_hbm.at[wid, pl.ds(sb * _SB, _SB)], dstb)

        def dstep(j, carry):
            pltpu.sync_copy(rows0, aggsh.at[dstb.at[j]], add=True)
            return carry

        lax.fori_loop(0, _SB, dstep, 0)
    plsc.subcore_barrier()
    pltpu.sync_copy(aggsh.at[pl.ds(r0, _RPT)], deg_hbm.at[c].at[pl.ds(r0, _RPT)])


_BLK = 1000


def _tc_body(h_ref, p_ref, d_ref, ws_ref, wn_ref, b_ref, o_ref):
    deg = d_ref[0, :, 0:1] + d_ref[1, :, 0:1]
    inv = 1.0 / jnp.maximum(deg, 1.0)
    mean = (p_ref[0] + p_ref[1]) * inv
    acc = jnp.dot(h_ref[...], ws_ref[...], preferred_element_type=jnp.float32)
    acc = acc + jnp.dot(mean, wn_ref[...], preferred_element_type=jnp.float32)
    o_ref[...] = jnp.maximum(acc + b_ref[...], 0.0)


_tc_layer = pl.pallas_call(
    _tc_body,
    grid=(_N // _BLK,),
    in_specs=[
        pl.BlockSpec((_BLK, _D), lambda i: (i, 0)),
        pl.BlockSpec((_NC, _BLK, _D), lambda i: (0, i, 0)),
        pl.BlockSpec((_NC, _BLK, _D), lambda i: (0, i, 0)),
        pl.BlockSpec((_D, _D), lambda i: (0, 0)),
        pl.BlockSpec((_D, _D), lambda i: (0, 0)),
        pl.BlockSpec((1, _D), lambda i: (0, 0)),
    ],
    out_specs=pl.BlockSpec((_BLK, _D), lambda i: (i, 0)),
    out_shape=jax.ShapeDtypeStruct((_N, _D), jnp.float32),
)


def kernel(x, edge_index, W_self_0, W_neigh_0, b_0, W_self_1, W_neigh_1, b_1,
           W_self_2, W_neigh_2, b_2):
    pad_src = (jnp.arange(_PAD, dtype=jnp.int32) * 41) % _N
    pad_dst = _N + jnp.arange(_PAD, dtype=jnp.int32)
    src = jnp.concatenate(
        [edge_index[0].reshape(_NW, _EPW),
         jnp.broadcast_to(pad_src, (_NW, _PAD))], axis=1
    ).reshape(_NW, _NCHUNK, _CH)
    dst = jnp.concatenate(
        [edge_index[1].reshape(_NW, _EPW),
         jnp.broadcast_to(pad_dst, (_NW, _PAD))], axis=1
    ).reshape(_NW, _NCHUNK, _CH)
    zeros = jnp.zeros((_NP, _D), jnp.float32)
    ones = jnp.ones((_CH, _D), jnp.float32)

    parts, degp = _sc_agg0(x, src, dst, zeros, ones)
    params = [(W_self_1, W_neigh_1, b_1), (W_self_2, W_neigh_2, b_2)]
    h = _tc_layer(x, parts, degp, W_self_0, W_neigh_0, b_0.reshape(1, _D))
    for Ws, Wn, b in params:
        parts = _sc_agg(h, src, dst, zeros)
        h = _tc_layer(h, parts, degp, Ws, Wn, b.reshape(1, _D))
    return h.reshape(1, _N, _D)


# speedup vs baseline: 9.9877x; 1.0149x over previous
import functools

import jax
import jax.numpy as jnp
from jax import lax
from jax.experimental import pallas as pl
from jax.experimental.pallas import tpu as pltpu
from jax.experimental.pallas import tpu_sc as plsc

_N = 10000
_D = 128
_E = 320000
_NC = 2
_NS = 16
_NW = _NC * _NS
_EPW = _E // _NW
_CH = 128
_NCHUNK = 80
_PAD = _NCHUNK * _CH - _EPW
_SB = 40
_NSB = _NCHUNK // _SB
_NP = 10240
_RPT = _NP // _NS

_mesh = plsc.VectorSubcoreMesh(core_axis_name="c", subcore_axis_name="s")

_scratch = [
    pltpu.VMEM((_SB, _CH), jnp.int32),
    pltpu.VMEM((_SB, _CH), jnp.int32),
    pltpu.VMEM((_CH, _D), jnp.float32),
    pltpu.VMEM((_CH, _D), jnp.float32),
    pltpu.VMEM_SHARED((_NP, _D), jnp.float32),
    pltpu.SemaphoreType.DMA,
    pltpu.SemaphoreType.DMA,
    pltpu.SemaphoreType.DMA,
    pltpu.SemaphoreType.DMA,
]


def _edge_pass(h_hbm, src_hbm, dst_hbm, wid, srcb, dstb, rows, gsems, ssems,
               aggsh):
    for sb in range(_NSB):
        pltpu.sync_copy(src_hbm.at[wid, pl.ds(sb * _SB, _SB)], srcb)
        pltpu.sync_copy(dst_hbm.at[wid, pl.ds(sb * _SB, _SB)], dstb)
        pltpu.async_copy(h_hbm.at[srcb.at[0]], rows[0], gsems[0])

        def step(g, carry):
            for b in range(2):
                j = 2 * g + b
                pltpu.make_async_copy(h_hbm.at[srcb.at[0]], rows[b],
                                      gsems[b]).wait()
                pltpu.async_copy(rows[b], aggsh.at[dstb.at[j]], ssems[b],
                                 add=True)

                @pl.when(j >= 1)
                def _():
                    pltpu.make_async_copy(rows[1 - b], aggsh.at[dstb.at[0]],
                                          ssems[1 - b]).wait()

                @pl.when(j + 1 < _SB)
                def _():
                    pltpu.async_copy(h_hbm.at[srcb.at[j + 1]], rows[1 - b],
                                     gsems[1 - b])

            return carry

        lax.fori_loop(0, _SB // 2, step, 0)
        pltpu.make_async_copy(rows[(_SB - 1) % 2], aggsh.at[dstb.at[0]],
                              ssems[(_SB - 1) % 2]).wait()


@functools.partial(
    pl.kernel,
    out_type=jax.ShapeDtypeStruct((_NC, _NP, _D), jnp.float32),
    mesh=_mesh,
    scratch_types=_scratch,
)
def _sc_agg(h_hbm, src_hbm, dst_hbm, zero_hbm, out_hbm,
            srcb, dstb, rows0, rows1, aggsh, gsem0, gsem1, ssem0, ssem1):
    c = lax.axis_index("c")
    s = lax.axis_index("s")
    wid = c * _NS + s
    r0 = s * _RPT
    pltpu.sync_copy(zero_hbm.at[pl.ds(r0, _RPT)], aggsh.at[pl.ds(r0, _RPT)])
    plsc.subcore_barrier()
    _edge_pass(h_hbm, src_hbm, dst_hbm, wid, srcb, dstb, (rows0, rows1),
               (gsem0, gsem1), (ssem0, ssem1), aggsh)
    plsc.subcore_barrier()
    pltpu.sync_copy(aggsh.at[pl.ds(r0, _RPT)], out_hbm.at[c].at[pl.ds(r0, _RPT)])


@functools.partial(
    pl.kernel,
    out_type=(jax.ShapeDtypeStruct((_NC, _NP, _D), jnp.float32),
              jax.ShapeDtypeStruct((_NC, _NP, _D), jnp.float32)),
    mesh=_mesh,
    scratch_types=_scratch,
)
def _sc_agg0(h_hbm, src_hbm, dst_hbm, zero_hbm, ones_hbm, out_hbm, deg_hbm,
             srcb, dstb, rows0, rows1, aggsh, gsem0, gsem1, ssem0, ssem1):
    c = lax.axis_index("c")
    s = lax.axis_index("s")
    wid = c * _NS + s
    r0 = s * _RPT
    pltpu.sync_copy(zero_hbm.at[pl.ds(r0, _RPT)], aggsh.at[pl.ds(r0, _RPT)])
    plsc.subcore_barrier()
    _edge_pass(h_hbm, src_hbm, dst_hbm, wid, srcb, dstb, (rows0, rows1),
               (gsem0, gsem1), (ssem0, ssem1), aggsh)
    plsc.subcore_barrier()
    pltpu.sync_copy(aggsh.at[pl.ds(r0, _RPT)], out_hbm.at[c].at[pl.ds(r0, _RPT)])
    pltpu.sync_copy(zero_hbm.at[pl.ds(r0, _RPT)], aggsh.at[pl.ds(r0, _RPT)])
    pltpu.sync_copy(ones_hbm, rows0)
    plsc.subcore_barrier()
    for sb in range(_NSB):
        pltpu.sync_copy(dst_hbm.at[wid, pl.ds(sb * _SB, _SB)], dstb)

        def dstep(j, carry):
            pltpu.sync_copy(rows0, aggsh.at[dstb.at[j]], add=True)
            return carry

        lax.fori_loop(0, _SB, dstep, 0)
    plsc.subcore_barrier()
    pltpu.sync_copy(aggsh.at[pl.ds(r0, _RPT)], deg_hbm.at[c].at[pl.ds(r0, _RPT)])


_BLK = 2000


def _tc_body(h_ref, p_ref, d_ref, ws_ref, wn_ref, b_ref, o_ref):
    deg = d_ref[0, :, 0:1] + d_ref[1, :, 0:1]
    inv = 1.0 / jnp.maximum(deg, 1.0)
    mean = (p_ref[0] + p_ref[1]) * inv
    acc = jnp.dot(h_ref[...], ws_ref[...], preferred_element_type=jnp.float32)
    acc = acc + jnp.dot(mean, wn_ref[...], preferred_element_type=jnp.float32)
    o_ref[...] = jnp.maximum(acc + b_ref[...], 0.0)


_tc_layer = pl.pallas_call(
    _tc_body,
    grid=(_N // _BLK,),
    in_specs=[
        pl.BlockSpec((_BLK, _D), lambda i: (i, 0)),
        pl.BlockSpec((_NC, _BLK, _D), lambda i: (0, i, 0)),
        pl.BlockSpec((_NC, _BLK, _D), lambda i: (0, i, 0)),
        pl.BlockSpec((_D, _D), lambda i: (0, 0)),
        pl.BlockSpec((_D, _D), lambda i: (0, 0)),
        pl.BlockSpec((1, _D), lambda i: (0, 0)),
    ],
    out_specs=pl.BlockSpec((_BLK, _D), lambda i: (i, 0)),
    out_shape=jax.ShapeDtypeStruct((_N, _D), jnp.float32),
)


def kernel(x, edge_index, W_self_0, W_neigh_0, b_0, W_self_1, W_neigh_1, b_1,
           W_self_2, W_neigh_2, b_2):
    pad_src = (jnp.arange(_PAD, dtype=jnp.int32) * 41) % _N
    pad_dst = _N + jnp.arange(_PAD, dtype=jnp.int32)
    src = jnp.concatenate(
        [edge_index[0].reshape(_NW, _EPW),
         jnp.broadcast_to(pad_src, (_NW, _PAD))], axis=1
    ).reshape(_NW, _NCHUNK, _CH)
    dst = jnp.concatenate(
        [edge_index[1].reshape(_NW, _EPW),
         jnp.broadcast_to(pad_dst, (_NW, _PAD))], axis=1
    ).reshape(_NW, _NCHUNK, _CH)
    zeros = jnp.zeros((_NP, _D), jnp.float32)
    ones = jnp.ones((_CH, _D), jnp.float32)

    parts, degp = _sc_agg0(x, src, dst, zeros, ones)
    params = [(W_self_1, W_neigh_1, b_1), (W_self_2, W_neigh_2, b_2)]
    h = _tc_layer(x, parts, degp, W_self_0, W_neigh_0, b_0.reshape(1, _D))
    for Ws, Wn, b in params:
        parts = _sc_agg(h, src, dst, zeros)
        h = _tc_layer(h, parts, degp, Ws, Wn, b.reshape(1, _D))
    return h.reshape(1, _N, _D)


# speedup vs baseline: 9.9993x; 1.0012x over previous
import functools

import jax
import jax.numpy as jnp
from jax import lax
from jax.experimental import pallas as pl
from jax.experimental.pallas import tpu as pltpu
from jax.experimental.pallas import tpu_sc as plsc

_N = 10000
_D = 128
_E = 320000
_NC = 2
_NS = 16
_NW = _NC * _NS
_EPW = _E // _NW
_CH = 128
_NCHUNK = 80
_PAD = _NCHUNK * _CH - _EPW
_SB = 40
_NSB = _NCHUNK // _SB
_NP = 10240
_RPT = _NP // _NS

_mesh = plsc.VectorSubcoreMesh(core_axis_name="c", subcore_axis_name="s")

_scratch = [
    pltpu.VMEM((_SB, _CH), jnp.int32),
    pltpu.VMEM((_SB, _CH), jnp.int32),
    pltpu.VMEM((_CH, _D), jnp.float32),
    pltpu.VMEM((_CH, _D), jnp.float32),
    pltpu.VMEM_SHARED((_NP, _D), jnp.float32),
    pltpu.SemaphoreType.DMA,
    pltpu.SemaphoreType.DMA,
    pltpu.SemaphoreType.DMA,
    pltpu.SemaphoreType.DMA,
]


def _edge_pass(h_hbm, src_hbm, dst_hbm, wid, srcb, dstb, rows, gsems, ssems,
               aggsh):
    for sb in range(_NSB):
        pltpu.sync_copy(src_hbm.at[wid, pl.ds(sb * _SB, _SB)], srcb)
        pltpu.sync_copy(dst_hbm.at[wid, pl.ds(sb * _SB, _SB)], dstb)
        pltpu.async_copy(h_hbm.at[srcb.at[0]], rows[0], gsems[0])

        def step(g, carry):
            for b in range(2):
                j = 2 * g + b
                pltpu.make_async_copy(h_hbm.at[srcb.at[0]], rows[b],
                                      gsems[b]).wait()
                pltpu.async_copy(rows[b], aggsh.at[dstb.at[j]], ssems[b],
                                 add=True)

                @pl.when(j >= 1)
                def _():
                    pltpu.make_async_copy(rows[1 - b], aggsh.at[dstb.at[0]],
                                          ssems[1 - b]).wait()

                @pl.when(j + 1 < _SB)
                def _():
                    pltpu.async_copy(h_hbm.at[srcb.at[j + 1]], rows[1 - b],
                                     gsems[1 - b])

            return carry

        lax.fori_loop(0, _SB // 2, step, 0)
        pltpu.make_async_copy(rows[(_SB - 1) % 2], aggsh.at[dstb.at[0]],
                              ssems[(_SB - 1) % 2]).wait()


@functools.partial(
    pl.kernel,
    out_type=jax.ShapeDtypeStruct((_NC, _NP, _D), jnp.float32),
    mesh=_mesh,
    scratch_types=_scratch,
)
def _sc_agg(h_hbm, src_hbm, dst_hbm, zero_hbm, out_hbm,
            srcb, dstb, rows0, rows1, aggsh, gsem0, gsem1, ssem0, ssem1):
    c = lax.axis_index("c")
    s = lax.axis_index("s")
    wid = c * _NS + s
    r0 = s * _RPT
    pltpu.sync_copy(zero_hbm.at[pl.ds(r0, _RPT)], aggsh.at[pl.ds(r0, _RPT)])
    plsc.subcore_barrier()
    _edge_pass(h_hbm, src_hbm, dst_hbm, wid, srcb, dstb, (rows0, rows1),
               (gsem0, gsem1), (ssem0, ssem1), aggsh)
    plsc.subcore_barrier()
    pltpu.sync_copy(aggsh.at[pl.ds(r0, _RPT)], out_hbm.at[c].at[pl.ds(r0, _RPT)])


@functools.partial(
    pl.kernel,
    out_type=(jax.ShapeDtypeStruct((_NC, _NP, _D), jnp.float32),
              jax.ShapeDtypeStruct((_NC, _NP, _D), jnp.float32)),
    mesh=_mesh,
    scratch_types=_scratch,
)
def _sc_agg0(h_hbm, src_hbm, dst_hbm, zero_hbm, ones_hbm, out_hbm, deg_hbm,
             srcb, dstb, rows0, rows1, aggsh, gsem0, gsem1, ssem0, ssem1):
    c = lax.axis_index("c")
    s = lax.axis_index("s")
    wid = c * _NS + s
    r0 = s * _RPT
    pltpu.sync_copy(zero_hbm.at[pl.ds(r0, _RPT)], aggsh.at[pl.ds(r0, _RPT)])
    plsc.subcore_barrier()
    _edge_pass(h_hbm, src_hbm, dst_hbm, wid, srcb, dstb, (rows0, rows1),
               (gsem0, gsem1), (ssem0, ssem1), aggsh)
    plsc.subcore_barrier()
    pltpu.sync_copy(aggsh.at[pl.ds(r0, _RPT)], out_hbm.at[c].at[pl.ds(r0, _RPT)])
    pltpu.sync_copy(zero_hbm.at[pl.ds(r0, _RPT)], aggsh.at[pl.ds(r0, _RPT)])
    pltpu.sync_copy(ones_hbm, rows0)
    plsc.subcore_barrier()
    for sb in range(_NSB - 1, -1, -1):
        if sb != _NSB - 1:
            pltpu.sync_copy(dst_hbm.at[wid, pl.ds(sb * _SB, _SB)], dstb)
        for j0 in range(4):
            pltpu.async_copy(rows0, aggsh.at[dstb.at[j0]], ssem0, add=True)

        def dstep(g, carry):
            pltpu.make_async_copy(rows0, aggsh.at[dstb.at[0]], ssem0).wait()
            pltpu.async_copy(rows0, aggsh.at[dstb.at[g + 4]], ssem0, add=True)
            return carry

        lax.fori_loop(0, _SB - 4, dstep, 0)
        for _ in range(4):
            pltpu.make_async_copy(rows0, aggsh.at[dstb.at[0]], ssem0).wait()
    plsc.subcore_barrier()
    pltpu.sync_copy(aggsh.at[pl.ds(r0, _RPT)], deg_hbm.at[c].at[pl.ds(r0, _RPT)])


_BLK = 2000


def _tc_body(h_ref, p_ref, d_ref, ws_ref, wn_ref, b_ref, o_ref):
    deg = d_ref[0, :, 0:1] + d_ref[1, :, 0:1]
    inv = 1.0 / jnp.maximum(deg, 1.0)
    mean = (p_ref[0] + p_ref[1]) * inv
    acc = jnp.dot(h_ref[...], ws_ref[...], preferred_element_type=jnp.float32)
    acc = acc + jnp.dot(mean, wn_ref[...], preferred_element_type=jnp.float32)
    o_ref[...] = jnp.maximum(acc + b_ref[...], 0.0)


_tc_layer = pl.pallas_call(
    _tc_body,
    grid=(_N // _BLK,),
    in_specs=[
        pl.BlockSpec((_BLK, _D), lambda i: (i, 0)),
        pl.BlockSpec((_NC, _BLK, _D), lambda i: (0, i, 0)),
        pl.BlockSpec((_NC, _BLK, _D), lambda i: (0, i, 0)),
        pl.BlockSpec((_D, _D), lambda i: (0, 0)),
        pl.BlockSpec((_D, _D), lambda i: (0, 0)),
        pl.BlockSpec((1, _D), lambda i: (0, 0)),
    ],
    out_specs=pl.BlockSpec((_BLK, _D), lambda i: (i, 0)),
    out_shape=jax.ShapeDtypeStruct((_N, _D), jnp.float32),
)


def kernel(x, edge_index, W_self_0, W_neigh_0, b_0, W_self_1, W_neigh_1, b_1,
           W_self_2, W_neigh_2, b_2):
    pad_src = (jnp.arange(_PAD, dtype=jnp.int32) * 41) % _N
    pad_dst = _N + jnp.arange(_PAD, dtype=jnp.int32)
    src = jnp.concatenate(
        [edge_index[0].reshape(_NW, _EPW),
         jnp.broadcast_to(pad_src, (_NW, _PAD))], axis=1
    ).reshape(_NW, _NCHUNK, _CH)
    dst = jnp.concatenate(
        [edge_index[1].reshape(_NW, _EPW),
         jnp.broadcast_to(pad_dst, (_NW, _PAD))], axis=1
    ).reshape(_NW, _NCHUNK, _CH)
    zeros = jnp.zeros((_NP, _D), jnp.float32)
    ones = jnp.ones((_CH, _D), jnp.float32)

    parts, degp = _sc_agg0(x, src, dst, zeros, ones)
    params = [(W_self_1, W_neigh_1, b_1), (W_self_2, W_neigh_2, b_2)]
    h = _tc_layer(x, parts, degp, W_self_0, W_neigh_0, b_0.reshape(1, _D))
    for Ws, Wn, b in params:
        parts = _sc_agg(h, src, dst, zeros)
        h = _tc_layer(h, parts, degp, Ws, Wn, b.reshape(1, _D))
    return h.reshape(1, _N, _D)


# speedup vs baseline: 10.0028x; 1.0004x over previous
import functools

import jax
import jax.numpy as jnp
from jax import lax
from jax.experimental import pallas as pl
from jax.experimental.pallas import tpu as pltpu
from jax.experimental.pallas import tpu_sc as plsc

_N = 10000
_D = 128
_E = 320000
_NC = 2
_NS = 16
_NW = _NC * _NS
_EPW = _E // _NW
_CH = 128
_NCHUNK = 80
_PAD = _NCHUNK * _CH - _EPW
_SB = 40
_NSB = _NCHUNK // _SB
_NP = 10240
_RPT = _NP // _NS

_mesh = plsc.VectorSubcoreMesh(core_axis_name="c", subcore_axis_name="s")

_scratch = [
    pltpu.VMEM((_SB, _CH), jnp.int32),
    pltpu.VMEM((_SB, _CH), jnp.int32),
    pltpu.VMEM((_CH, _D), jnp.float32),
    pltpu.VMEM((_CH, _D), jnp.float32),
    pltpu.VMEM_SHARED((_NP, _D), jnp.float32),
    pltpu.SemaphoreType.DMA,
    pltpu.SemaphoreType.DMA,
    pltpu.SemaphoreType.DMA,
    pltpu.SemaphoreType.DMA,
]


def _edge_pass(h_hbm, src_hbm, dst_hbm, wid, srcb, dstb, rows, gsems, ssems,
               aggsh):
    for sb in range(_NSB):
        pltpu.sync_copy(src_hbm.at[wid, pl.ds(sb * _SB, _SB)], srcb)
        pltpu.sync_copy(dst_hbm.at[wid, pl.ds(sb * _SB, _SB)], dstb)
        pltpu.async_copy(h_hbm.at[srcb.at[0]], rows[0], gsems[0])

        def step(g, carry):
            for b in range(2):
                j = 2 * g + b
                pltpu.make_async_copy(h_hbm.at[srcb.at[0]], rows[b],
                                      gsems[b]).wait()
                pltpu.async_copy(rows[b], aggsh.at[dstb.at[j]], ssems[b],
                                 add=True)

                @pl.when(j >= 1)
                def _():
                    pltpu.make_async_copy(rows[1 - b], aggsh.at[dstb.at[0]],
                                          ssems[1 - b]).wait()

                @pl.when(j + 1 < _SB)
                def _():
                    pltpu.async_copy(h_hbm.at[srcb.at[j + 1]], rows[1 - b],
                                     gsems[1 - b])

            return carry

        lax.fori_loop(0, _SB // 2, step, 0)
        pltpu.make_async_copy(rows[(_SB - 1) % 2], aggsh.at[dstb.at[0]],
                              ssems[(_SB - 1) % 2]).wait()


@functools.partial(
    pl.kernel,
    out_type=jax.ShapeDtypeStruct((_NC, _NP, _D), jnp.float32),
    mesh=_mesh,
    scratch_types=_scratch,
)
def _sc_agg(h_hbm, src_hbm, dst_hbm, zero_hbm, out_hbm,
            srcb, dstb, rows0, rows1, aggsh, gsem0, gsem1, ssem0, ssem1):
    c = lax.axis_index("c")
    s = lax.axis_index("s")
    wid = c * _NS + s
    r0 = s * _RPT
    pltpu.sync_copy(zero_hbm.at[pl.ds(r0, _RPT)], aggsh.at[pl.ds(r0, _RPT)])
    plsc.subcore_barrier()
    _edge_pass(h_hbm, src_hbm, dst_hbm, wid, srcb, dstb, (rows0, rows1),
               (gsem0, gsem1), (ssem0, ssem1), aggsh)
    plsc.subcore_barrier()
    pltpu.sync_copy(aggsh.at[pl.ds(r0, _RPT)], out_hbm.at[c].at[pl.ds(r0, _RPT)])


@functools.partial(
    pl.kernel,
    out_type=(jax.ShapeDtypeStruct((_NC, _NP, _D), jnp.float32),
              jax.ShapeDtypeStruct((_NC, _NP, _D), jnp.float32)),
    mesh=_mesh,
    scratch_types=_scratch,
)
def _sc_agg0(h_hbm, src_hbm, dst_hbm, zero_hbm, ones_hbm, out_hbm, deg_hbm,
             srcb, dstb, rows0, rows1, aggsh, gsem0, gsem1, ssem0, ssem1):
    c = lax.axis_index("c")
    s = lax.axis_index("s")
    wid = c * _NS + s
    r0 = s * _RPT
    pltpu.sync_copy(zero_hbm.at[pl.ds(r0, _RPT)], aggsh.at[pl.ds(r0, _RPT)])
    plsc.subcore_barrier()
    _edge_pass(h_hbm, src_hbm, dst_hbm, wid, srcb, dstb, (rows0, rows1),
               (gsem0, gsem1), (ssem0, ssem1), aggsh)
    plsc.subcore_barrier()
    pltpu.sync_copy(aggsh.at[pl.ds(r0, _RPT)], out_hbm.at[c].at[pl.ds(r0, _RPT)])
    pltpu.sync_copy(zero_hbm.at[pl.ds(r0, _RPT)], aggsh.at[pl.ds(r0, _RPT)])
    pltpu.sync_copy(ones_hbm, rows0)
    plsc.subcore_barrier()
    for sb in range(_NSB - 1, -1, -1):
        if sb != _NSB - 1:
            pltpu.sync_copy(dst_hbm.at[wid, pl.ds(sb * _SB, _SB)], dstb)
        for j0 in range(4):
            pltpu.async_copy(rows0, aggsh.at[dstb.at[j0]], ssem0, add=True)

        def dstep(g, carry):
            pltpu.make_async_copy(rows0, aggsh.at[dstb.at[0]], ssem0).wait()
            pltpu.async_copy(rows0, aggsh.at[dstb.at[g + 4]], ssem0, add=True)
            return carry

        lax.fori_loop(0, _SB - 4, dstep, 0)
        for _ in range(4):
            pltpu.make_async_copy(rows0, aggsh.at[dstb.at[0]], ssem0).wait()
    plsc.subcore_barrier()
    pltpu.sync_copy(aggsh.at[pl.ds(r0, _RPT)], deg_hbm.at[c].at[pl.ds(r0, _RPT)])


_BLK = 2000


def _tc0_body(h_ref, p_ref, d_ref, ws_ref, wn_ref, b_ref, o_ref, iv_ref):
    deg = d_ref[0, :, 0:1] + d_ref[1, :, 0:1]
    inv = 1.0 / jnp.maximum(deg, 1.0)
    iv_ref[...] = jnp.broadcast_to(inv, (_BLK, 8))
    mean = (p_ref[0] + p_ref[1]) * inv
    acc = jnp.dot(h_ref[...], ws_ref[...], preferred_element_type=jnp.float32)
    acc = acc + jnp.dot(mean, wn_ref[...], preferred_element_type=jnp.float32)
    o_ref[...] = jnp.maximum(acc + b_ref[...], 0.0)


_tc_layer0 = pl.pallas_call(
    _tc0_body,
    grid=(_N // _BLK,),
    in_specs=[
        pl.BlockSpec((_BLK, _D), lambda i: (i, 0)),
        pl.BlockSpec((_NC, _BLK, _D), lambda i: (0, i, 0)),
        pl.BlockSpec((_NC, _BLK, _D), lambda i: (0, i, 0)),
        pl.BlockSpec((_D, _D), lambda i: (0, 0)),
        pl.BlockSpec((_D, _D), lambda i: (0, 0)),
        pl.BlockSpec((1, _D), lambda i: (0, 0)),
    ],
    out_specs=[
        pl.BlockSpec((_BLK, _D), lambda i: (i, 0)),
        pl.BlockSpec((_BLK, 8), lambda i: (i, 0)),
    ],
    out_shape=[
        jax.ShapeDtypeStruct((_N, _D), jnp.float32),
        jax.ShapeDtypeStruct((_N, 8), jnp.float32),
    ],
)


def _tc_body(h_ref, p_ref, iv_ref, ws_ref, wn_ref, b_ref, o_ref):
    mean = (p_ref[0] + p_ref[1]) * iv_ref[:, 0:1]
    acc = jnp.dot(h_ref[...], ws_ref[...], preferred_element_type=jnp.float32)
    acc = acc + jnp.dot(mean, wn_ref[...], preferred_element_type=jnp.float32)
    o_ref[...] = jnp.maximum(acc + b_ref[...], 0.0)


_tc_layer = pl.pallas_call(
    _tc_body,
    grid=(_N // _BLK,),
    in_specs=[
        pl.BlockSpec((_BLK, _D), lambda i: (i, 0)),
        pl.BlockSpec((_NC, _BLK, _D), lambda i: (0, i, 0)),
        pl.BlockSpec((_BLK, 8), lambda i: (i, 0)),
        pl.BlockSpec((_D, _D), lambda i: (0, 0)),
        pl.BlockSpec((_D, _D), lambda i: (0, 0)),
        pl.BlockSpec((1, _D), lambda i: (0, 0)),
    ],
    out_specs=pl.BlockSpec((_BLK, _D), lambda i: (i, 0)),
    out_shape=jax.ShapeDtypeStruct((_N, _D), jnp.float32),
)


def kernel(x, edge_index, W_self_0, W_neigh_0, b_0, W_self_1, W_neigh_1, b_1,
           W_self_2, W_neigh_2, b_2):
    pad_src = (jnp.arange(_PAD, dtype=jnp.int32) * 41) % _N
    pad_dst = _N + jnp.arange(_PAD, dtype=jnp.int32)
    src = jnp.concatenate(
        [edge_index[0].reshape(_NW, _EPW),
         jnp.broadcast_to(pad_src, (_NW, _PAD))], axis=1
    ).reshape(_NW, _NCHUNK, _CH)
    dst = jnp.concatenate(
        [edge_index[1].reshape(_NW, _EPW),
         jnp.broadcast_to(pad_dst, (_NW, _PAD))], axis=1
    ).reshape(_NW, _NCHUNK, _CH)
    zeros = jnp.zeros((_NP, _D), jnp.float32)
    ones = jnp.ones((_CH, _D), jnp.float32)

    parts, degp = _sc_agg0(x, src, dst, zeros, ones)
    params = [(W_self_1, W_neigh_1, b_1), (W_self_2, W_neigh_2, b_2)]
    h, invd = _tc_layer0(x, parts, degp, W_self_0, W_neigh_0,
                         b_0.reshape(1, _D))
    for Ws, Wn, b in params:
        parts = _sc_agg(h, src, dst, zeros)
        h = _tc_layer(h, parts, invd, Ws, Wn, b.reshape(1, _D))
    return h.reshape(1, _N, _D)
